# Initial kernel scaffold; baseline (speedup 1.0000x reference)
import jax
import jax.numpy as jnp
from jax.experimental import pallas as pl


def kernel(x, edge_index_rel0, edge_index_rel1, W1_0, al1_0, ar1_0, b1_0, W1_1, al1_1, ar1_1, b1_1, W2_0, al2_0, ar2_0, b2_0, W2_1, al2_1, ar2_1, b2_1, W3_0, b3_0, W3_1, b3_1):
    raise NotImplementedError("write your pallas kernel here")


# speedup vs baseline: 1.0275x; 1.0275x over previous
import jax
import jax.numpy as jnp
from jax.experimental import pallas as pl

N = 10000
E = 160000
HID = 256
HEADS = 2


def _mm_kernel(a_ref, b_ref, o_ref):
    o_ref[...] = jnp.dot(a_ref[...], b_ref[...], preferred_element_type=jnp.float32)


def _mm(a, b, bm=400):
    m, k = a.shape
    k2, n = b.shape
    return pl.pallas_call(
        _mm_kernel,
        grid=(m // bm,),
        in_specs=[
            pl.BlockSpec((bm, k), lambda i: (i, 0)),
            pl.BlockSpec((k, n), lambda i: (0, 0)),
        ],
        out_specs=pl.BlockSpec((bm, n), lambda i: (i, 0)),
        out_shape=jax.ShapeDtypeStruct((m, n), jnp.float32),
    )(a, b)


def _gat(x, src, dst, W, a_l, a_r, b):
    h = _mm(x, W).reshape(N, HEADS, HID)
    el = (h * a_l[None]).sum(-1)
    er = (h * a_r[None]).sum(-1)
    e = jax.nn.leaky_relu(el[src] + er[dst], negative_slope=0.2)
    ee = jnp.exp(e)
    denom = jax.ops.segment_sum(ee, dst, num_segments=N)
    alpha = ee / (denom[dst] + 1e-9)
    out = jax.ops.segment_sum(h[src] * alpha[:, :, None], dst, num_segments=N)
    return out + b[None]


def _gcn(x, src, dst, W, b):
    ones = jnp.ones((src.shape[0],), dtype=jnp.float32)
    deg_out = jnp.clip(jax.ops.segment_sum(ones, src, num_segments=N), 1.0, None)
    deg_in = jnp.clip(jax.ops.segment_sum(ones, dst, num_segments=N), 1.0, None)
    h = x * (deg_out ** -0.5)[:, None]
    agg = jax.ops.segment_sum(h[src], dst, num_segments=N)
    agg = agg * (deg_in ** -0.5)[:, None]
    return _mm(agg, W) + b


def kernel(x, edge_index_rel0, edge_index_rel1, W1_0, al1_0, ar1_0, b1_0, W1_1, al1_1, ar1_1, b1_1, W2_0, al2_0, ar2_0, b2_0, W2_1, al2_1, ar2_1, b2_1, W3_0, b3_0, W3_1, b3_1):
    s0, d0 = edge_index_rel0[0], edge_index_rel0[1]
    s1, d1 = edge_index_rel1[0], edge_index_rel1[1]
    h = x
    att = _gat(h, s0, d0, W1_0, al1_0, ar1_0, b1_0) + _gat(h, s1, d1, W1_1, al1_1, ar1_1, b1_1)
    h = jnp.mean(jax.nn.relu(att), axis=1)
    att = _gat(h, s0, d0, W2_0, al2_0, ar2_0, b2_0) + _gat(h, s1, d1, W2_1, al2_1, ar2_1, b2_1)
    h = jnp.mean(jax.nn.relu(att), axis=1)
    out = _gcn(h, s0, d0, W3_0, b3_0) + _gcn(h, s1, d1, W3_1, b3_1)
    return out


# speedup vs baseline: 11.3110x; 11.0080x over previous
import functools

import jax
import jax.numpy as jnp
from jax import lax
from jax.experimental import pallas as pl
from jax.experimental.pallas import tpu as pltpu
from jax.experimental.pallas import tpu_sc as plsc

N = 10000
E = 160000
HID = 256
HEADS = 2
NCLS = 128

NW = 32
RT = 320
NP = NW * RT
B = 64
EP = E + B
NB = 64

f32 = jnp.float32
i32 = jnp.int32

_MESH = plsc.VectorSubcoreMesh(core_axis_name="c", subcore_axis_name="s")


def _wid():
    return lax.axis_index("s") * 2 + lax.axis_index("c")


def _zero_vec():
    return jnp.zeros((16,), f32)


def _zero_rows(ref, nrows):
    zz = _zero_vec()

    def zrow(r, carry):
        for k in range(HID // 16):
            ref[r, pl.ds(16 * k, 16)] = zz
        return carry

    lax.fori_loop(0, nrows, zrow, 0)


def _zero_flat(ref, nwords):
    zz = _zero_vec()

    def zchunk(i, carry):
        ref[pl.ds(16 * i, 16)] = zz
        return carry

    lax.fori_loop(0, nwords // 16, zchunk, 0)


def _leaky(v):
    return jnp.where(v > 0, v, 0.2 * v)


def _gat_sc_body(srcd, dstd, bounds, el0, el1, er0, er1, h0, h1,
                 out0, out1,
                 bnd_v, el0_t, el1_t, er0_l, er1_l, den0, den1,
                 src_b, dst_b, dlo_b, alp_b, stage, acc, sem):
    base = _wid() * RT
    pltpu.sync_copy(bounds, bnd_v)
    pltpu.sync_copy(el0, el0_t)
    pltpu.sync_copy(el1, el1_t)
    pltpu.sync_copy(er0.at[pl.ds(base, RT)], er0_l)
    pltpu.sync_copy(er1.at[pl.ds(base, RT)], er1_l)
    bv = bnd_v[pl.ds(_wid(), 16)]
    start = bv[0]
    end = bv[1]
    start_al = (start // 8) * 8
    nblk = (end - start_al + B - 1) // B

    _zero_flat(den0, RT)
    _zero_flat(den1, RT)

    iot = lax.iota(i32, 16)

    def _edge_vecs(s0, g):
        sv = src_b[pl.ds(16 * g, 16)]
        dv = dst_b[pl.ds(16 * g, 16)]
        gi = s0 + 16 * g + iot
        valid = (gi >= start) & (gi < end)
        dl = jnp.clip(dv - base, 0, RT - 1)
        return sv, dl, valid

    def p1_blk(b, carry):
        s0 = pl.multiple_of(start_al + b * B, 8)
        pltpu.sync_copy(srcd.at[pl.ds(s0, B)], src_b)
        pltpu.sync_copy(dstd.at[pl.ds(s0, B)], dst_b)
        for g in range(B // 16):
            sv, dl, valid = _edge_vecs(s0, g)
            e0 = plsc.load_gather(el0_t, [sv]) + plsc.load_gather(er0_l, [dl])
            e1 = plsc.load_gather(el1_t, [sv]) + plsc.load_gather(er1_l, [dl])
            ee0 = jnp.where(valid, jnp.exp(_leaky(e0)), 0.0)
            ee1 = jnp.where(valid, jnp.exp(_leaky(e1)), 0.0)
            plsc.addupdate_scatter(den0, [dl], ee0)
            plsc.addupdate_scatter(den1, [dl], ee1)
        return carry

    lax.fori_loop(0, nblk, p1_blk, 0)

    def agg_pass(el_t, er_l, den, h_hbm, out_hbm):
        _zero_rows(acc, RT)

        def blk(b, carry):
            s0 = pl.multiple_of(start_al + b * B, 8)
            pltpu.sync_copy(srcd.at[pl.ds(s0, B)], src_b)
            pltpu.sync_copy(dstd.at[pl.ds(s0, B)], dst_b)
            pltpu.async_copy(h_hbm.at[src_b], stage, sem).wait()
            for g in range(B // 16):
                sv, dl, valid = _edge_vecs(s0, g)
                e = plsc.load_gather(el_t, [sv]) + plsc.load_gather(er_l, [dl])
                ee = jnp.exp(_leaky(e))
                dn = plsc.load_gather(den, [dl])
                a = jnp.where(valid, ee / (dn + 1e-9), 0.0)
                alp_b[pl.ds(16 * g, 16)] = a
                dlo_b[pl.ds(16 * g, 16)] = dl

            def edge(j, c2):
                dj = dlo_b[pl.ds(j, 16)][0]
                aj = alp_b[pl.ds(j, 16)][0]
                for k in range(HID // 16):
                    plsc.addupdate(acc.at[dj, pl.ds(16 * k, 16)],
                                   aj * stage[j, pl.ds(16 * k, 16)])
                return c2

            lax.fori_loop(0, B, edge, 0)
            return carry

        lax.fori_loop(0, nblk, blk, 0)
        pltpu.sync_copy(acc, out_hbm.at[pl.ds(base, RT)])

    agg_pass(el0_t, er0_l, den0, h0, out0)
    agg_pass(el1_t, er1_l, den1, h1, out1)


_gat_sc = pl.kernel(
    _gat_sc_body,
    out_type=[jax.ShapeDtypeStruct((NP, HID), f32),
              jax.ShapeDtypeStruct((NP, HID), f32)],
    mesh=_MESH,
    compiler_params=pltpu.CompilerParams(needs_layout_passes=False),
    scratch_types=[
        pltpu.VMEM((NB,), i32),
        pltpu.VMEM((NP,), f32),
        pltpu.VMEM((NP,), f32),
        pltpu.VMEM((RT,), f32),
        pltpu.VMEM((RT,), f32),
        pltpu.VMEM((RT,), f32),
        pltpu.VMEM((RT,), f32),
        pltpu.VMEM((B,), i32),
        pltpu.VMEM((B,), i32),
        pltpu.VMEM((B + 16,), i32),
        pltpu.VMEM((B + 16,), f32),
        pltpu.VMEM((B, HID), f32),
        pltpu.VMEM((RT, HID), f32),
        pltpu.SemaphoreType.DMA,
    ],
)


def _deg_sc_body(srcs, bounds, deg, bnd_v, cnt, src_b):
    base = _wid() * RT
    pltpu.sync_copy(bounds, bnd_v)
    bv = bnd_v[pl.ds(_wid(), 16)]
    start = bv[0]
    end = bv[1]
    start_al = (start // 8) * 8
    nblk = (end - start_al + B - 1) // B
    _zero_flat(cnt, RT)
    iot = lax.iota(i32, 16)
    one = jnp.ones((16,), f32)

    def blk(b, carry):
        s0 = pl.multiple_of(start_al + b * B, 8)
        pltpu.sync_copy(srcs.at[pl.ds(s0, B)], src_b)
        for g in range(B // 16):
            sv = src_b[pl.ds(16 * g, 16)]
            gi = s0 + 16 * g + iot
            valid = (gi >= start) & (gi < end)
            sl = jnp.clip(sv - base, 0, RT - 1)
            plsc.addupdate_scatter(cnt, [sl], jnp.where(valid, one, 0.0))
        return carry

    lax.fori_loop(0, nblk, blk, 0)
    pltpu.sync_copy(cnt, deg.at[pl.ds(base, RT)])


_deg_sc = pl.kernel(
    _deg_sc_body,
    out_type=jax.ShapeDtypeStruct((NP,), f32),
    mesh=_MESH,
    compiler_params=pltpu.CompilerParams(needs_layout_passes=False),
    scratch_types=[
        pltpu.VMEM((NB,), i32),
        pltpu.VMEM((RT,), f32),
        pltpu.VMEM((B,), i32),
    ],
)


def _gcn_sc_body(srcd, dstd, bounds, dinv, h,
                 deg_in, agg,
                 bnd_v, dinv_t, cnt, src_b, dst_b, dlo_b, w_b, stage, acc, sem):
    base = _wid() * RT
    pltpu.sync_copy(bounds, bnd_v)
    pltpu.sync_copy(dinv, dinv_t)
    bv = bnd_v[pl.ds(_wid(), 16)]
    start = bv[0]
    end = bv[1]
    start_al = (start // 8) * 8
    nblk = (end - start_al + B - 1) // B
    _zero_flat(cnt, RT)
    _zero_rows(acc, RT)
    iot = lax.iota(i32, 16)
    one = jnp.ones((16,), f32)

    def blk(b, carry):
        s0 = pl.multiple_of(start_al + b * B, 8)
        pltpu.sync_copy(srcd.at[pl.ds(s0, B)], src_b)
        pltpu.sync_copy(dstd.at[pl.ds(s0, B)], dst_b)
        pltpu.async_copy(h.at[src_b], stage, sem).wait()
        for g in range(B // 16):
            sv = src_b[pl.ds(16 * g, 16)]
            dv = dst_b[pl.ds(16 * g, 16)]
            gi = s0 + 16 * g + iot
            valid = (gi >= start) & (gi < end)
            dl = jnp.clip(dv - base, 0, RT - 1)
            wv = jnp.where(valid, plsc.load_gather(dinv_t, [sv]), 0.0)
            plsc.addupdate_scatter(cnt, [dl], jnp.where(valid, one, 0.0))
            w_b[pl.ds(16 * g, 16)] = wv
            dlo_b[pl.ds(16 * g, 16)] = dl

        def edge(j, c2):
            dj = dlo_b[pl.ds(j, 16)][0]
            wj = w_b[pl.ds(j, 16)][0]
            for k in range(HID // 16):
                plsc.addupdate(acc.at[dj, pl.ds(16 * k, 16)],
                               wj * stage[j, pl.ds(16 * k, 16)])
            return c2

        lax.fori_loop(0, B, edge, 0)
        return carry

    lax.fori_loop(0, nblk, blk, 0)
    pltpu.sync_copy(cnt, deg_in.at[pl.ds(base, RT)])
    pltpu.sync_copy(acc, agg.at[pl.ds(base, RT)])


_gcn_sc = pl.kernel(
    _gcn_sc_body,
    out_type=[jax.ShapeDtypeStruct((NP,), f32),
              jax.ShapeDtypeStruct((NP, HID), f32)],
    mesh=_MESH,
    compiler_params=pltpu.CompilerParams(needs_layout_passes=False),
    scratch_types=[
        pltpu.VMEM((NB,), i32),
        pltpu.VMEM((NP,), f32),
        pltpu.VMEM((RT,), f32),
        pltpu.VMEM((B,), i32),
        pltpu.VMEM((B,), i32),
        pltpu.VMEM((B + 16,), i32),
        pltpu.VMEM((B + 16,), f32),
        pltpu.VMEM((B, HID), f32),
        pltpu.VMEM((RT, HID), f32),
        pltpu.SemaphoreType.DMA,
    ],
)


def _k1_body(x_ref, w_ref, al_ref, ar_ref,
             h0_ref, h1_ref, el0_ref, el1_ref, er0_ref, er1_ref):
    h = jnp.dot(x_ref[...], w_ref[...], preferred_element_type=f32)
    h0 = h[:, :HID]
    h1 = h[:, HID:]
    h0_ref[...] = h0
    h1_ref[...] = h1
    al = al_ref[...]
    ar = ar_ref[...]
    el0_ref[...] = jnp.sum(h0 * al[0][None, :], axis=1, keepdims=True)
    el1_ref[...] = jnp.sum(h1 * al[1][None, :], axis=1, keepdims=True)
    er0_ref[...] = jnp.sum(h0 * ar[0][None, :], axis=1, keepdims=True)
    er1_ref[...] = jnp.sum(h1 * ar[1][None, :], axis=1, keepdims=True)


def _k1(x, w, al, ar):
    grid = (NP // RT,)
    return pl.pallas_call(
        _k1_body,
        grid=grid,
        in_specs=[
            pl.BlockSpec((RT, HID), lambda i: (i, 0)),
            pl.BlockSpec((HID, HEADS * HID), lambda i: (0, 0)),
            pl.BlockSpec((HEADS, HID), lambda i: (0, 0)),
            pl.BlockSpec((HEADS, HID), lambda i: (0, 0)),
        ],
        out_specs=[
            pl.BlockSpec((RT, HID), lambda i: (i, 0)),
            pl.BlockSpec((RT, HID), lambda i: (i, 0)),
            pl.BlockSpec((RT, 1), lambda i: (i, 0)),
            pl.BlockSpec((RT, 1), lambda i: (i, 0)),
            pl.BlockSpec((RT, 1), lambda i: (i, 0)),
            pl.BlockSpec((RT, 1), lambda i: (i, 0)),
        ],
        out_shape=[
            jax.ShapeDtypeStruct((NP, HID), f32),
            jax.ShapeDtypeStruct((NP, HID), f32),
            jax.ShapeDtypeStruct((NP, 1), f32),
            jax.ShapeDtypeStruct((NP, 1), f32),
            jax.ShapeDtypeStruct((NP, 1), f32),
            jax.ShapeDtypeStruct((NP, 1), f32),
        ],
    )(x, w, al, ar)


def _k2_body(o00_ref, o01_ref, o10_ref, o11_ref, b_ref, x_ref):
    b = b_ref[...]
    att0 = o00_ref[...] + o10_ref[...] + b[0][None, :]
    att1 = o01_ref[...] + o11_ref[...] + b[1][None, :]
    x_ref[...] = 0.5 * (jnp.maximum(att0, 0.0) + jnp.maximum(att1, 0.0))


def _k2(o00, o01, o10, o11, bsum):
    return pl.pallas_call(
        _k2_body,
        grid=(NP // RT,),
        in_specs=[pl.BlockSpec((RT, HID), lambda i: (i, 0))] * 4
        + [pl.BlockSpec((HEADS, HID), lambda i: (0, 0))],
        out_specs=pl.BlockSpec((RT, HID), lambda i: (i, 0)),
        out_shape=jax.ShapeDtypeStruct((NP, HID), f32),
    )(o00, o01, o10, o11, bsum)


def _kdinv_body(d0_ref, d1_ref, o0_ref, o1_ref):
    o0_ref[...] = jax.lax.rsqrt(jnp.maximum(d0_ref[...], 1.0))
    o1_ref[...] = jax.lax.rsqrt(jnp.maximum(d1_ref[...], 1.0))


def _kdinv(d0, d1):
    return pl.pallas_call(
        _kdinv_body,
        grid=(NP // RT,),
        in_specs=[pl.BlockSpec((RT, 1), lambda i: (i, 0))] * 2,
        out_specs=[pl.BlockSpec((RT, 1), lambda i: (i, 0))] * 2,
        out_shape=[jax.ShapeDtypeStruct((NP, 1), f32)] * 2,
    )(d0, d1)


def _k3_body(a0_ref, di0_ref, w0_ref, a1_ref, di1_ref, w1_ref, b_ref, o_ref):
    s0 = jax.lax.rsqrt(jnp.maximum(di0_ref[...], 1.0))
    s1 = jax.lax.rsqrt(jnp.maximum(di1_ref[...], 1.0))
    acc = jnp.dot(a0_ref[...] * s0, w0_ref[...], preferred_element_type=f32)
    acc += jnp.dot(a1_ref[...] * s1, w1_ref[...], preferred_element_type=f32)
    o_ref[...] = acc + b_ref[...]


def _k3(a0, di0, w0, a1, di1, w1, bsum):
    return pl.pallas_call(
        _k3_body,
        grid=(NP // RT,),
        in_specs=[
            pl.BlockSpec((RT, HID), lambda i: (i, 0)),
            pl.BlockSpec((RT, 1), lambda i: (i, 0)),
            pl.BlockSpec((HID, NCLS), lambda i: (0, 0)),
            pl.BlockSpec((RT, HID), lambda i: (i, 0)),
            pl.BlockSpec((RT, 1), lambda i: (i, 0)),
            pl.BlockSpec((HID, NCLS), lambda i: (0, 0)),
            pl.BlockSpec((1, NCLS), lambda i: (0, 0)),
        ],
        out_specs=pl.BlockSpec((RT, NCLS), lambda i: (i, 0)),
        out_shape=jax.ShapeDtypeStruct((NP, NCLS), f32),
    )(a0, di0, w0, a1, di1, w1, bsum)


def _prep_edges(edge_index):
    s, d = edge_index[0], edge_index[1]
    order = jnp.argsort(d)
    ds = d[order]
    ss = s[order]
    marks = (jnp.arange(NW + 1, dtype=i32) * RT)
    bounds = jnp.searchsorted(ds, marks).astype(i32)
    bounds = jnp.concatenate([bounds, jnp.zeros((NB - NW - 1,), i32)])
    pad = jnp.zeros((B,), i32)
    srcd = jnp.concatenate([ss, pad])
    dstd = jnp.concatenate([ds, pad])
    sso = jnp.sort(s)
    bounds_s = jnp.searchsorted(sso, marks).astype(i32)
    bounds_s = jnp.concatenate([bounds_s, jnp.zeros((NB - NW - 1,), i32)])
    srcs = jnp.concatenate([sso, pad])
    return srcd, dstd, bounds, srcs, bounds_s


def kernel(x, edge_index_rel0, edge_index_rel1, W1_0, al1_0, ar1_0, b1_0, W1_1, al1_1, ar1_1, b1_1, W2_0, al2_0, ar2_0, b2_0, W2_1, al2_1, ar2_1, b2_1, W3_0, b3_0, W3_1, b3_1):
    srcd0, dstd0, bnd0, srcs0, bnds0 = _prep_edges(edge_index_rel0)
    srcd1, dstd1, bnd1, srcs1, bnds1 = _prep_edges(edge_index_rel1)

    xp = jnp.pad(x, ((0, NP - N), (0, 0)))

    def gat_layer(xin, W_0, al_0, ar_0, W_1, al_1, ar_1, bsum):
        h0_0, h1_0, el0_0, el1_0, er0_0, er1_0 = _k1(xin, W_0, al_0, ar_0)
        h0_1, h1_1, el0_1, el1_1, er0_1, er1_1 = _k1(xin, W_1, al_1, ar_1)
        o0_0, o1_0 = _gat_sc(srcd0, dstd0, bnd0,
                             el0_0.reshape(NP), el1_0.reshape(NP),
                             er0_0.reshape(NP), er1_0.reshape(NP), h0_0, h1_0)
        o0_1, o1_1 = _gat_sc(srcd1, dstd1, bnd1,
                             el0_1.reshape(NP), el1_1.reshape(NP),
                             er0_1.reshape(NP), er1_1.reshape(NP), h0_1, h1_1)
        return _k2(o0_0, o1_0, o0_1, o1_1, bsum)

    h = gat_layer(xp, W1_0, al1_0, ar1_0, W1_1, al1_1, ar1_1, b1_0 + b1_1)
    h = gat_layer(h, W2_0, al2_0, ar2_0, W2_1, al2_1, ar2_1, b2_0 + b2_1)

    dego0 = _deg_sc(srcs0, bnds0)
    dego1 = _deg_sc(srcs1, bnds1)
    dinv0, dinv1 = _kdinv(dego0.reshape(NP, 1), dego1.reshape(NP, 1))
    din0, agg0 = _gcn_sc(srcd0, dstd0, bnd0, dinv0.reshape(NP), h)
    din1, agg1 = _gcn_sc(srcd1, dstd1, bnd1, dinv1.reshape(NP), h)
    out = _k3(agg0, din0.reshape(NP, 1), W3_0,
              agg1, din1.reshape(NP, 1), W3_1,
              (b3_0 + b3_1).reshape(1, NCLS))
    return out[:N]


# speedup vs baseline: 13.9517x; 1.2335x over previous
import jax
import jax.numpy as jnp
from jax import lax
from jax.experimental import pallas as pl
from jax.experimental.pallas import tpu as pltpu
from jax.experimental.pallas import tpu_sc as plsc

N = 10000
E = 160000
HID = 256
HEADS = 2
NCLS = 128

NW = 32
RT = 320
NP = NW * RT
B = 64
EP = E + 8 * B
NB = 64

f32 = jnp.float32
i32 = jnp.int32

_MESH = plsc.VectorSubcoreMesh(core_axis_name="c", subcore_axis_name="s")
_SC_PARAMS = pltpu.CompilerParams(needs_layout_passes=False)


def _wid():
    return lax.axis_index("s") * 2 + lax.axis_index("c")


def _zero_rows(ref, nrows):
    zz = jnp.zeros((16,), f32)

    def zrow(r, carry):
        for k in range(HID // 16):
            ref[r, pl.ds(16 * k, 16)] = zz
        return carry

    lax.fori_loop(0, nrows, zrow, 0)


def _zero_flat(ref, nwords):
    zz = jnp.zeros((16,), f32)

    def zchunk(i, carry):
        ref[pl.ds(16 * i, 16)] = zz
        return carry

    lax.fori_loop(0, nwords // 16, zchunk, 0)


def _leaky(v):
    return jnp.where(v > 0, v, 0.2 * v)


def _tile_range(bnd_v):
    bv = bnd_v[pl.ds(_wid(), 16)]
    start = bv[0]
    end = bv[1]
    start_al = (start // 8) * 8
    nblk = (end - start_al + B - 1) // B
    return start, end, start_al, nblk


def _gat_sc_body(srcd, dstd, bounds, el0, el1, er0, er1, h0, h1,
                 out0, out1,
                 bnd_v, el_t, er_l, den,
                 src_0, src_1, dst_0, dst_1, dlo_b, alp_b,
                 stage_0, stage_1, acc,
                 sem_i0, sem_i1, sem_r0, sem_r1):
    base = _wid() * RT
    pltpu.sync_copy(bounds, bnd_v)
    start, end, start_al, nblk = _tile_range(bnd_v)
    npair = (nblk + 1) // 2

    srcs = [src_0, src_1]
    dsts = [dst_0, dst_1]
    stages = [stage_0, stage_1]
    semi = [sem_i0, sem_i1]
    semr = [sem_r0, sem_r1]
    iot = lax.iota(i32, 16)

    def s0_of(b):
        return pl.multiple_of(start_al + b * B, 8)

    def issue_idx(b, k):
        s0 = s0_of(b)
        pltpu.async_copy(srcd.at[pl.ds(s0, B)], srcs[k], semi[k])
        pltpu.async_copy(dstd.at[pl.ds(s0, B)], dsts[k], semi[k])

    def wait_idx(k):
        pltpu.make_async_copy(srcd.at[pl.ds(0, B)], srcs[k], semi[k]).wait()
        pltpu.make_async_copy(dstd.at[pl.ds(0, B)], dsts[k], semi[k]).wait()

    def issue_row(h_hbm, k):
        pltpu.async_copy(h_hbm.at[srcs[k]], stages[k], semr[k])

    def wait_row(h_hbm, k):
        pltpu.make_async_copy(h_hbm.at[srcs[k]], stages[k], semr[k]).wait()

    def edge_group(k, s0, g):
        sv = srcs[k][pl.ds(16 * g, 16)]
        dv = dsts[k][pl.ds(16 * g, 16)]
        gi = s0 + 16 * g + iot
        valid = (gi >= start) & (gi < end)
        dl = jnp.clip(dv - base, 0, RT - 1)
        return sv, dl, valid

    def den_pass(el_hbm, er_hbm):
        pltpu.sync_copy(el_hbm, el_t)
        pltpu.sync_copy(er_hbm.at[pl.ds(base, RT)], er_l)
        _zero_flat(den, RT)
        issue_idx(0, 0)
        issue_idx(1, 1)

        def pair(p, carry):
            for u in range(2):
                k = u
                b = 2 * p + u
                wait_idx(k)
                s0 = s0_of(b)
                for g in range(B // 16):
                    sv, dl, valid = edge_group(k, s0, g)
                    e = plsc.load_gather(el_t, [sv]) + plsc.load_gather(er_l, [dl])
                    ee = jnp.where(valid, jnp.exp(_leaky(e)), 0.0)
                    plsc.addupdate_scatter(den, [dl], ee)
                issue_idx(b + 2, k)
            return carry

        lax.fori_loop(0, npair, pair, 0)
        wait_idx(0)
        wait_idx(1)

    def agg_pass(h_hbm, out_hbm):
        _zero_rows(acc, RT)
        issue_idx(0, 0)
        wait_idx(0)
        issue_row(h_hbm, 0)
        issue_idx(1, 1)

        def pair(p, carry):
            for u in range(2):
                cur = u
                nxt = 1 - u
                b = 2 * p + u
                wait_idx(nxt)
                issue_row(h_hbm, nxt)
                wait_row(h_hbm, cur)
                s0 = s0_of(b)
                for g in range(B // 16):
                    sv, dl, valid = edge_group(cur, s0, g)
                    e = plsc.load_gather(el_t, [sv]) + plsc.load_gather(er_l, [dl])
                    ee = jnp.exp(_leaky(e))
                    dn = plsc.load_gather(den, [dl])
                    a = jnp.where(valid, ee / (dn + 1e-9), 0.0)
                    alp_b[pl.ds(16 * g, 16)] = a
                    dlo_b[pl.ds(16 * g, 16)] = dl

                def edge(j, c2):
                    dj = dlo_b[pl.ds(j, 16)][0]
                    aj = alp_b[pl.ds(j, 16)][0]
                    st = stages[cur]
                    for kk in range(HID // 16):
                        plsc.addupdate(acc.at[dj, pl.ds(16 * kk, 16)],
                                       aj * st[j, pl.ds(16 * kk, 16)])
                    return c2

                lax.fori_loop(0, B, edge, 0)
                issue_idx(b + 2, cur)
            return carry

        lax.fori_loop(0, npair, pair, 0)
        wait_row(h_hbm, 0)
        wait_idx(1)
        pltpu.sync_copy(acc, out_hbm.at[pl.ds(base, RT)])

    den_pass(el0, er0)
    agg_pass(h0, out0)
    den_pass(el1, er1)
    agg_pass(h1, out1)


_gat_sc = pl.kernel(
    _gat_sc_body,
    out_type=[jax.ShapeDtypeStruct((NP, HID), f32),
              jax.ShapeDtypeStruct((NP, HID), f32)],
    mesh=_MESH,
    compiler_params=_SC_PARAMS,
    scratch_types=[
        pltpu.VMEM((NB,), i32),
        pltpu.VMEM((NP,), f32),
        pltpu.VMEM((RT,), f32),
        pltpu.VMEM((RT,), f32),
        pltpu.VMEM((B,), i32),
        pltpu.VMEM((B,), i32),
        pltpu.VMEM((B,), i32),
        pltpu.VMEM((B,), i32),
        pltpu.VMEM((B + 16,), i32),
        pltpu.VMEM((B + 16,), f32),
        pltpu.VMEM((B, HID), f32),
        pltpu.VMEM((B, HID), f32),
        pltpu.VMEM((RT, HID), f32),
        pltpu.SemaphoreType.DMA,
        pltpu.SemaphoreType.DMA,
        pltpu.SemaphoreType.DMA,
        pltpu.SemaphoreType.DMA,
    ],
)


def _deg_sc_body(srcs_hbm, bounds, deg, bnd_v, cnt, src_0, src_1,
                 sem_i0, sem_i1):
    base = _wid() * RT
    pltpu.sync_copy(bounds, bnd_v)
    start, end, start_al, nblk = _tile_range(bnd_v)
    npair = (nblk + 1) // 2
    srcs = [src_0, src_1]
    semi = [sem_i0, sem_i1]
    _zero_flat(cnt, RT)
    iot = lax.iota(i32, 16)
    one = jnp.ones((16,), f32)

    def issue_idx(b, k):
        s0 = pl.multiple_of(start_al + b * B, 8)
        pltpu.async_copy(srcs_hbm.at[pl.ds(s0, B)], srcs[k], semi[k])

    def wait_idx(k):
        pltpu.make_async_copy(srcs_hbm.at[pl.ds(0, B)], srcs[k], semi[k]).wait()

    issue_idx(0, 0)
    issue_idx(1, 1)

    def pair(p, carry):
        for u in range(2):
            k = u
            b = 2 * p + u
            wait_idx(k)
            s0 = pl.multiple_of(start_al + b * B, 8)
            for g in range(B // 16):
                sv = srcs[k][pl.ds(16 * g, 16)]
                gi = s0 + 16 * g + iot
                valid = (gi >= start) & (gi < end)
                sl = jnp.clip(sv - base, 0, RT - 1)
                plsc.addupdate_scatter(cnt, [sl], jnp.where(valid, one, 0.0))
            issue_idx(b + 2, k)
        return carry

    lax.fori_loop(0, npair, pair, 0)
    wait_idx(0)
    wait_idx(1)
    pltpu.sync_copy(cnt, deg.at[pl.ds(base, RT)])


_deg_sc = pl.kernel(
    _deg_sc_body,
    out_type=jax.ShapeDtypeStruct((NP,), f32),
    mesh=_MESH,
    compiler_params=_SC_PARAMS,
    scratch_types=[
        pltpu.VMEM((NB,), i32),
        pltpu.VMEM((RT,), f32),
        pltpu.VMEM((B,), i32),
        pltpu.VMEM((B,), i32),
        pltpu.SemaphoreType.DMA,
        pltpu.SemaphoreType.DMA,
    ],
)


def _gcn_sc_body(srcd, dstd, bounds, dinv, h,
                 deg_in, agg,
                 bnd_v, dinv_t, cnt,
                 src_0, src_1, dst_0, dst_1, dlo_b, alp_b,
                 stage_0, stage_1, acc,
                 sem_i0, sem_i1, sem_r0, sem_r1):
    base = _wid() * RT
    pltpu.sync_copy(bounds, bnd_v)
    pltpu.sync_copy(dinv, dinv_t)
    start, end, start_al, nblk = _tile_range(bnd_v)
    npair = (nblk + 1) // 2

    srcs = [src_0, src_1]
    dsts = [dst_0, dst_1]
    stages = [stage_0, stage_1]
    semi = [sem_i0, sem_i1]
    semr = [sem_r0, sem_r1]
    iot = lax.iota(i32, 16)
    one = jnp.ones((16,), f32)

    _zero_flat(cnt, RT)
    _zero_rows(acc, RT)

    def issue_idx(b, k):
        s0 = pl.multiple_of(start_al + b * B, 8)
        pltpu.async_copy(srcd.at[pl.ds(s0, B)], srcs[k], semi[k])
        pltpu.async_copy(dstd.at[pl.ds(s0, B)], dsts[k], semi[k])

    def wait_idx(k):
        pltpu.make_async_copy(srcd.at[pl.ds(0, B)], srcs[k], semi[k]).wait()
        pltpu.make_async_copy(dstd.at[pl.ds(0, B)], dsts[k], semi[k]).wait()

    def issue_row(k):
        pltpu.async_copy(h.at[srcs[k]], stages[k], semr[k])

    def wait_row(k):
        pltpu.make_async_copy(h.at[srcs[k]], stages[k], semr[k]).wait()

    issue_idx(0, 0)
    wait_idx(0)
    issue_row(0)
    issue_idx(1, 1)

    def pair(p, carry):
        for u in range(2):
            cur = u
            nxt = 1 - u
            b = 2 * p + u
            wait_idx(nxt)
            issue_row(nxt)
            wait_row(cur)
            s0 = pl.multiple_of(start_al + b * B, 8)
            for g in range(B // 16):
                sv = srcs[cur][pl.ds(16 * g, 16)]
                dv = dsts[cur][pl.ds(16 * g, 16)]
                gi = s0 + 16 * g + iot
                valid = (gi >= start) & (gi < end)
                dl = jnp.clip(dv - base, 0, RT - 1)
                wv = jnp.where(valid, plsc.load_gather(dinv_t, [sv]), 0.0)
                plsc.addupdate_scatter(cnt, [dl], jnp.where(valid, one, 0.0))
                alp_b[pl.ds(16 * g, 16)] = wv
                dlo_b[pl.ds(16 * g, 16)] = dl

            def edge(j, c2):
                dj = dlo_b[pl.ds(j, 16)][0]
                wj = alp_b[pl.ds(j, 16)][0]
                st = stages[cur]
                for kk in range(HID // 16):
                    plsc.addupdate(acc.at[dj, pl.ds(16 * kk, 16)],
                                   wj * st[j, pl.ds(16 * kk, 16)])
                return c2

            lax.fori_loop(0, B, edge, 0)
            issue_idx(b + 2, cur)
        return carry

    lax.fori_loop(0, npair, pair, 0)
    wait_row(0)
    wait_idx(1)
    pltpu.sync_copy(cnt, deg_in.at[pl.ds(base, RT)])
    pltpu.sync_copy(acc, agg.at[pl.ds(base, RT)])


_gcn_sc = pl.kernel(
    _gcn_sc_body,
    out_type=[jax.ShapeDtypeStruct((NP,), f32),
              jax.ShapeDtypeStruct((NP, HID), f32)],
    mesh=_MESH,
    compiler_params=_SC_PARAMS,
    scratch_types=[
        pltpu.VMEM((NB,), i32),
        pltpu.VMEM((NP,), f32),
        pltpu.VMEM((RT,), f32),
        pltpu.VMEM((B,), i32),
        pltpu.VMEM((B,), i32),
        pltpu.VMEM((B,), i32),
        pltpu.VMEM((B,), i32),
        pltpu.VMEM((B + 16,), i32),
        pltpu.VMEM((B + 16,), f32),
        pltpu.VMEM((B, HID), f32),
        pltpu.VMEM((B, HID), f32),
        pltpu.VMEM((RT, HID), f32),
        pltpu.SemaphoreType.DMA,
        pltpu.SemaphoreType.DMA,
        pltpu.SemaphoreType.DMA,
        pltpu.SemaphoreType.DMA,
    ],
)


def _k1_body(x_ref, w_ref, al_ref, ar_ref,
             h0_ref, h1_ref, el0_ref, el1_ref, er0_ref, er1_ref):
    h = jnp.dot(x_ref[...], w_ref[...], preferred_element_type=f32)
    h0 = h[:, :HID]
    h1 = h[:, HID:]
    h0_ref[...] = h0
    h1_ref[...] = h1
    al = al_ref[...]
    ar = ar_ref[...]
    el0_ref[...] = jnp.sum(h0 * al[0][None, :], axis=1, keepdims=True)
    el1_ref[...] = jnp.sum(h1 * al[1][None, :], axis=1, keepdims=True)
    er0_ref[...] = jnp.sum(h0 * ar[0][None, :], axis=1, keepdims=True)
    er1_ref[...] = jnp.sum(h1 * ar[1][None, :], axis=1, keepdims=True)


def _k1(x, w, al, ar):
    return pl.pallas_call(
        _k1_body,
        grid=(NP // RT,),
        in_specs=[
            pl.BlockSpec((RT, HID), lambda i: (i, 0)),
            pl.BlockSpec((HID, HEADS * HID), lambda i: (0, 0)),
            pl.BlockSpec((HEADS, HID), lambda i: (0, 0)),
            pl.BlockSpec((HEADS, HID), lambda i: (0, 0)),
        ],
        out_specs=[
            pl.BlockSpec((RT, HID), lambda i: (i, 0)),
            pl.BlockSpec((RT, HID), lambda i: (i, 0)),
            pl.BlockSpec((RT, 1), lambda i: (i, 0)),
            pl.BlockSpec((RT, 1), lambda i: (i, 0)),
            pl.BlockSpec((RT, 1), lambda i: (i, 0)),
            pl.BlockSpec((RT, 1), lambda i: (i, 0)),
        ],
        out_shape=[
            jax.ShapeDtypeStruct((NP, HID), f32),
            jax.ShapeDtypeStruct((NP, HID), f32),
            jax.ShapeDtypeStruct((NP, 1), f32),
            jax.ShapeDtypeStruct((NP, 1), f32),
            jax.ShapeDtypeStruct((NP, 1), f32),
            jax.ShapeDtypeStruct((NP, 1), f32),
        ],
    )(x, w, al, ar)


def _k2_body(o00_ref, o01_ref, o10_ref, o11_ref, b_ref, x_ref):
    b = b_ref[...]
    att0 = o00_ref[...] + o10_ref[...] + b[0][None, :]
    att1 = o01_ref[...] + o11_ref[...] + b[1][None, :]
    x_ref[...] = 0.5 * (jnp.maximum(att0, 0.0) + jnp.maximum(att1, 0.0))


def _k2(o00, o01, o10, o11, bsum):
    return pl.pallas_call(
        _k2_body,
        grid=(NP // RT,),
        in_specs=[pl.BlockSpec((RT, HID), lambda i: (i, 0))] * 4
        + [pl.BlockSpec((HEADS, HID), lambda i: (0, 0))],
        out_specs=pl.BlockSpec((RT, HID), lambda i: (i, 0)),
        out_shape=jax.ShapeDtypeStruct((NP, HID), f32),
    )(o00, o01, o10, o11, bsum)


def _kdinv_body(d0_ref, d1_ref, o0_ref, o1_ref):
    o0_ref[...] = jax.lax.rsqrt(jnp.maximum(d0_ref[...], 1.0))
    o1_ref[...] = jax.lax.rsqrt(jnp.maximum(d1_ref[...], 1.0))


def _kdinv(d0, d1):
    return pl.pallas_call(
        _kdinv_body,
        grid=(NP // RT,),
        in_specs=[pl.BlockSpec((RT, 1), lambda i: (i, 0))] * 2,
        out_specs=[pl.BlockSpec((RT, 1), lambda i: (i, 0))] * 2,
        out_shape=[jax.ShapeDtypeStruct((NP, 1), f32)] * 2,
    )(d0, d1)


def _k3_body(a0_ref, di0_ref, w0_ref, a1_ref, di1_ref, w1_ref, b_ref, o_ref):
    s0 = jax.lax.rsqrt(jnp.maximum(di0_ref[...], 1.0))
    s1 = jax.lax.rsqrt(jnp.maximum(di1_ref[...], 1.0))
    acc = jnp.dot(a0_ref[...] * s0, w0_ref[...], preferred_element_type=f32)
    acc += jnp.dot(a1_ref[...] * s1, w1_ref[...], preferred_element_type=f32)
    o_ref[...] = acc + b_ref[...]


def _k3(a0, di0, w0, a1, di1, w1, bsum):
    return pl.pallas_call(
        _k3_body,
        grid=(NP // RT,),
        in_specs=[
            pl.BlockSpec((RT, HID), lambda i: (i, 0)),
            pl.BlockSpec((RT, 1), lambda i: (i, 0)),
            pl.BlockSpec((HID, NCLS), lambda i: (0, 0)),
            pl.BlockSpec((RT, HID), lambda i: (i, 0)),
            pl.BlockSpec((RT, 1), lambda i: (i, 0)),
            pl.BlockSpec((HID, NCLS), lambda i: (0, 0)),
            pl.BlockSpec((1, NCLS), lambda i: (0, 0)),
        ],
        out_specs=pl.BlockSpec((RT, NCLS), lambda i: (i, 0)),
        out_shape=jax.ShapeDtypeStruct((NP, NCLS), f32),
    )(a0, di0, w0, a1, di1, w1, bsum)


def _prep_edges(edge_index):
    s, d = edge_index[0], edge_index[1]
    order = jnp.argsort(d)
    ds = d[order]
    ss = s[order]
    marks = (jnp.arange(NW + 1, dtype=i32) * RT)
    bounds = jnp.searchsorted(ds, marks).astype(i32)
    bounds = jnp.concatenate([bounds, jnp.zeros((NB - NW - 1,), i32)])
    pad = jnp.zeros((EP - E,), i32)
    srcd = jnp.concatenate([ss, pad])
    dstd = jnp.concatenate([ds, pad])
    sso = jnp.sort(s)
    bounds_s = jnp.searchsorted(sso, marks).astype(i32)
    bounds_s = jnp.concatenate([bounds_s, jnp.zeros((NB - NW - 1,), i32)])
    srcs = jnp.concatenate([sso, pad])
    return srcd, dstd, bounds, srcs, bounds_s


def kernel(x, edge_index_rel0, edge_index_rel1, W1_0, al1_0, ar1_0, b1_0, W1_1, al1_1, ar1_1, b1_1, W2_0, al2_0, ar2_0, b2_0, W2_1, al2_1, ar2_1, b2_1, W3_0, b3_0, W3_1, b3_1):
    srcd0, dstd0, bnd0, srcs0, bnds0 = _prep_edges(edge_index_rel0)
    srcd1, dstd1, bnd1, srcs1, bnds1 = _prep_edges(edge_index_rel1)

    xp = jnp.pad(x, ((0, NP - N), (0, 0)))

    def gat_layer(xin, W_0, al_0, ar_0, W_1, al_1, ar_1, bsum):
        h0_0, h1_0, el0_0, el1_0, er0_0, er1_0 = _k1(xin, W_0, al_0, ar_0)
        h0_1, h1_1, el0_1, el1_1, er0_1, er1_1 = _k1(xin, W_1, al_1, ar_1)
        o0_0, o1_0 = _gat_sc(srcd0, dstd0, bnd0,
                             el0_0.reshape(NP), el1_0.reshape(NP),
                             er0_0.reshape(NP), er1_0.reshape(NP), h0_0, h1_0)
        o0_1, o1_1 = _gat_sc(srcd1, dstd1, bnd1,
                             el0_1.reshape(NP), el1_1.reshape(NP),
                             er0_1.reshape(NP), er1_1.reshape(NP), h0_1, h1_1)
        return _k2(o0_0, o1_0, o0_1, o1_1, bsum)

    h = gat_layer(xp, W1_0, al1_0, ar1_0, W1_1, al1_1, ar1_1, b1_0 + b1_1)
    h = gat_layer(h, W2_0, al2_0, ar2_0, W2_1, al2_1, ar2_1, b2_0 + b2_1)

    dego0 = _deg_sc(srcs0, bnds0)
    dego1 = _deg_sc(srcs1, bnds1)
    dinv0, dinv1 = _kdinv(dego0.reshape(NP, 1), dego1.reshape(NP, 1))
    din0, agg0 = _gcn_sc(srcd0, dstd0, bnd0, dinv0.reshape(NP), h)
    din1, agg1 = _gcn_sc(srcd1, dstd1, bnd1, dinv1.reshape(NP), h)
    out = _k3(agg0, din0.reshape(NP, 1), W3_0,
              agg1, din1.reshape(NP, 1), W3_1,
              (b3_0 + b3_1).reshape(1, NCLS))
    return out[:N]
